# baseline (speedup 1.0000x reference)
import functools

import jax
import jax.numpy as jnp
from jax import lax
from jax.experimental import pallas as pl
from jax.experimental.pallas import tpu as pltpu
from jax.experimental.pallas import tpu_sc as plsc

B = 16384
NF = 26
VOCAB = 100000
D = 16
NCONT = 10

NW = 32
ROWS_W = B // NW
GR_W = ROWS_W * NF
QN = GR_W // 128
GQ = 13
NG = QN // GQ
CN = ROWS_W * NCONT


def _sc_body(tflat, xcat3, xcont, offs, cshift, cscale, emb, xd,
             idx_v, gbuf, cin_v, cout_v, offs_v, shift_v, scale_v, sem):
    wid = lax.axis_index("c") * 16 + lax.axis_index("s")

    pltpu.sync_copy(offs, offs_v)
    pltpu.sync_copy(cshift, shift_v)
    pltpu.sync_copy(cscale, scale_v)
    pltpu.sync_copy(xcat3.at[wid], idx_v)

    def add_offs(q, _):
        m = (q * 128) % 208
        for v in range(8):
            sl = pl.ds(v * 16, 16)
            idx_v[q, sl] = idx_v[q, sl] + offs_v[pl.ds(m + v * 16, 16)]
        return 0
    lax.fori_loop(0, QN, add_offs, 0)

    gbase = wid * GR_W

    def group(g, _):
        q0 = g * GQ
        cps = []
        for j in range(GQ):
            cps.append(pltpu.async_copy(
                tflat.at[idx_v.at[q0 + j]],
                gbuf.at[pl.ds(j * 128, 128)], sem))
        for cp in cps:
            cp.wait()
        pltpu.sync_copy(gbuf, emb.at[pl.ds(gbase + q0 * 128, GQ * 128)])
        return 0
    lax.fori_loop(0, NG, group, 0)

    cbase = wid * CN
    pltpu.sync_copy(xcont.at[pl.ds(cbase, CN)], cin_v)

    def cont(v, _):
        m = (v * 16) % 80
        sl = pl.ds(v * 16, 16)
        cout_v[sl] = (cin_v[sl] - shift_v[pl.ds(m, 16)]) * scale_v[pl.ds(m, 16)]
        return 0
    lax.fori_loop(0, CN // 16, cont, 0)
    pltpu.sync_copy(cout_v, xd.at[pl.ds(cbase, CN)])


@jax.jit
def _run(tflat, xcat3, xcont, offs, cshift, cscale):
    f = pl.kernel(
        _sc_body,
        out_type=(
            jax.ShapeDtypeStruct((B * NF, D), jnp.float32),
            jax.ShapeDtypeStruct((B * NCONT,), jnp.float32),
        ),
        mesh=plsc.VectorSubcoreMesh(core_axis_name="c", subcore_axis_name="s"),
        scratch_types=[
            pltpu.VMEM((QN, 128), jnp.int32),
            pltpu.VMEM((GQ * 128, D), jnp.float32),
            pltpu.VMEM((CN,), jnp.float32),
            pltpu.VMEM((CN,), jnp.float32),
            pltpu.VMEM((NF * 16,), jnp.int32),
            pltpu.VMEM((96,), jnp.float32),
            pltpu.VMEM((96,), jnp.float32),
            pltpu.SemaphoreType.DMA,
        ],
        compiler_params=pltpu.CompilerParams(use_tc_tiling_on_sc=False),
    )
    return f(tflat, xcat3, xcont, offs, cshift, cscale)


def kernel(x_cat, x_cont, tables, cont_min, cont_max):
    tflat = tables.reshape(NF * VOCAB, D)
    xcat3 = x_cat.astype(jnp.int32).reshape(NW, QN, 128)
    xcont = x_cont.reshape(B * NCONT)
    offs = jnp.tile(jnp.arange(NF, dtype=jnp.int32) * VOCAB, 16)
    cshift = jnp.tile(cont_min, 10)[:96]
    cscale = jnp.tile(1.0 / (cont_max - cont_min), 10)[:96]
    emb, xd = _run(tflat, xcat3, xcont, offs, cshift, cscale)
    return jnp.concatenate([emb.reshape(B, NF * D), xd.reshape(B, NCONT)],
                           axis=1)


# speedup vs baseline: 1.0014x; 1.0014x over previous
import jax
import jax.numpy as jnp
from jax import lax
from jax.experimental import pallas as pl
from jax.experimental.pallas import tpu as pltpu
from jax.experimental.pallas import tpu_sc as plsc

B = 16384
NF = 26
VOCAB = 100000
D = 16
NCONT = 10
NCOL = NF * D + NCONT

NW = 32
ROWS_W = B // NW
GR_W = ROWS_W * NF
NBUF = 4

TROWS = 256
CBLK = (NF * D) // 128


def _sc_body(tflat, xcat2, out,
             idx_v, b0, b1, b2, b3, gsem, osem):
    wid = lax.axis_index("c") * 16 + lax.axis_index("s")
    row0 = wid * ROWS_W

    pltpu.sync_copy(xcat2.at[wid], idx_v)

    bufs = [b0, b1, b2, b3]

    def fire_gather(f):
        return pltpu.async_copy(
            tflat.at[idx_v.at[pl.ds(f * ROWS_W, ROWS_W)]],
            bufs[f % NBUF], gsem)

    cps_g = {}
    cps_o = {}
    for f in range(NBUF - 1):
        cps_g[f] = fire_gather(f)

    for f in range(NF):
        cps_g[f].wait()
        cps_o[f] = pltpu.async_copy(
            bufs[f % NBUF],
            out.at[pl.ds(row0, ROWS_W), pl.ds(f * D, D)], osem)
        if f - 1 >= 0:
            cps_o[f - 1].wait()
        if f + NBUF - 1 < NF:
            cps_g[f + NBUF - 1] = fire_gather(f + NBUF - 1)
    cps_o[NF - 1].wait()


def _tc_patch(xc_ref, sh_ref, sc_ref, prev_ref, o_ref):
    cont = (xc_ref[...] - sh_ref[...]) * sc_ref[...]
    blk = prev_ref[...]
    o_ref[...] = blk
    o_ref[:, NF * D - CBLK * 128:NCOL - CBLK * 128] = cont


@jax.jit
def _run(tflat, xcat2, xcont, shift, scale):
    emb = pl.kernel(
        _sc_body,
        out_type=jax.ShapeDtypeStruct((B, NCOL), jnp.float32),
        mesh=plsc.VectorSubcoreMesh(core_axis_name="c", subcore_axis_name="s"),
        scratch_types=[
            pltpu.VMEM((GR_W,), jnp.int32),
            pltpu.VMEM((ROWS_W, D), jnp.float32),
            pltpu.VMEM((ROWS_W, D), jnp.float32),
            pltpu.VMEM((ROWS_W, D), jnp.float32),
            pltpu.VMEM((ROWS_W, D), jnp.float32),
            pltpu.SemaphoreType.DMA,
            pltpu.SemaphoreType.DMA,
        ],
        compiler_params=pltpu.CompilerParams(use_tc_tiling_on_sc=False),
    )(tflat, xcat2)

    return pl.pallas_call(
        _tc_patch,
        grid=(B // TROWS,),
        in_specs=[
            pl.BlockSpec((TROWS, NCONT), lambda i: (i, 0)),
            pl.BlockSpec((1, NCONT), lambda i: (0, 0)),
            pl.BlockSpec((1, NCONT), lambda i: (0, 0)),
            pl.BlockSpec((TROWS, 128), lambda i: (i, CBLK)),
        ],
        out_specs=pl.BlockSpec((TROWS, 128), lambda i: (i, CBLK)),
        out_shape=jax.ShapeDtypeStruct((B, NCOL), jnp.float32),
        input_output_aliases={3: 0},
    )(xcont, shift, scale, emb)


def kernel(x_cat, x_cont, tables, cont_min, cont_max):
    tflat = tables.reshape(NF * VOCAB, D)
    offs = jnp.arange(NF, dtype=jnp.int32) * VOCAB
    xcat2 = (x_cat.astype(jnp.int32) + offs).reshape(
        NW, ROWS_W, NF).transpose(0, 2, 1).reshape(NW, GR_W)
    shift = cont_min.reshape(1, NCONT)
    scale = (1.0 / (cont_max - cont_min)).reshape(1, NCONT)
    return _run(tflat, xcat2, x_cont, shift, scale)


# speedup vs baseline: 1.0032x; 1.0018x over previous
import jax
import jax.numpy as jnp
from jax import lax
from jax.experimental import pallas as pl
from jax.experimental.pallas import tpu as pltpu
from jax.experimental.pallas import tpu_sc as plsc

B = 16384
NF = 26
VOCAB = 100000
D = 16
NCONT = 10
NCOL = NF * D + NCONT

NW = 32
ROWS_W = B // NW
GR_W = ROWS_W * NF
NBUF = 8

TROWS = 256
CBLK = (NF * D) // 128


def _sc_body(tflat, xcat2, out,
             idx_v, b0, b1, b2, b3, b4, b5, b6, b7, gsem, osem):
    wid = lax.axis_index("c") * 16 + lax.axis_index("s")
    row0 = wid * ROWS_W

    pltpu.sync_copy(xcat2.at[wid], idx_v)

    bufs = [b0, b1, b2, b3, b4, b5, b6, b7]

    def fire_gather(f):
        return pltpu.async_copy(
            tflat.at[idx_v.at[pl.ds(f * ROWS_W, ROWS_W)]],
            bufs[f % NBUF], gsem)

    cps_g = {}
    cps_o = {}
    for f in range(NBUF - 1):
        cps_g[f] = fire_gather(f)

    for f in range(NF):
        cps_g[f].wait()
        cps_o[f] = pltpu.async_copy(
            bufs[f % NBUF],
            out.at[pl.ds(row0, ROWS_W), pl.ds(f * D, D)], osem)
        if f - 1 >= 0:
            cps_o[f - 1].wait()
        if f + NBUF - 1 < NF:
            cps_g[f + NBUF - 1] = fire_gather(f + NBUF - 1)
    cps_o[NF - 1].wait()


def _tc_patch(xc_ref, sh_ref, sc_ref, prev_ref, o_ref):
    cont = (xc_ref[...] - sh_ref[...]) * sc_ref[...]
    blk = prev_ref[...]
    o_ref[...] = blk
    o_ref[:, NF * D - CBLK * 128:NCOL - CBLK * 128] = cont


@jax.jit
def _run(tflat, xcat2, xcont, shift, scale):
    emb = pl.kernel(
        _sc_body,
        out_type=jax.ShapeDtypeStruct((B, NCOL), jnp.float32),
        mesh=plsc.VectorSubcoreMesh(core_axis_name="c", subcore_axis_name="s"),
        scratch_types=[
            pltpu.VMEM((GR_W,), jnp.int32),
            pltpu.VMEM((ROWS_W, D), jnp.float32),
            pltpu.VMEM((ROWS_W, D), jnp.float32),
            pltpu.VMEM((ROWS_W, D), jnp.float32),
            pltpu.VMEM((ROWS_W, D), jnp.float32),
            pltpu.VMEM((ROWS_W, D), jnp.float32),
            pltpu.VMEM((ROWS_W, D), jnp.float32),
            pltpu.VMEM((ROWS_W, D), jnp.float32),
            pltpu.VMEM((ROWS_W, D), jnp.float32),
            pltpu.SemaphoreType.DMA,
            pltpu.SemaphoreType.DMA,
        ],
        compiler_params=pltpu.CompilerParams(use_tc_tiling_on_sc=False),
    )(tflat, xcat2)

    return pl.pallas_call(
        _tc_patch,
        grid=(B // TROWS,),
        in_specs=[
            pl.BlockSpec((TROWS, NCONT), lambda i: (i, 0)),
            pl.BlockSpec((1, NCONT), lambda i: (0, 0)),
            pl.BlockSpec((1, NCONT), lambda i: (0, 0)),
            pl.BlockSpec((TROWS, 128), lambda i: (i, CBLK)),
        ],
        out_specs=pl.BlockSpec((TROWS, 128), lambda i: (i, CBLK)),
        out_shape=jax.ShapeDtypeStruct((B, NCOL), jnp.float32),
        input_output_aliases={3: 0},
    )(xcont, shift, scale, emb)


def kernel(x_cat, x_cont, tables, cont_min, cont_max):
    tflat = tables.reshape(NF * VOCAB, D)
    offs = jnp.arange(NF, dtype=jnp.int32) * VOCAB
    xcat2 = (x_cat.astype(jnp.int32) + offs).reshape(
        NW, ROWS_W, NF).transpose(0, 2, 1).reshape(NW, GR_W)
    shift = cont_min.reshape(1, NCONT)
    scale = (1.0 / (cont_max - cont_min)).reshape(1, NCONT)
    return _run(tflat, xcat2, x_cont, shift, scale)
